# baseline (speedup 1.0000x reference)
import functools

import jax
import jax.numpy as jnp
from jax import lax
from jax.experimental import pallas as pl
from jax.experimental.pallas import tpu as pltpu
from jax.experimental.pallas import tpu_sc as plsc

C = 1000
CPAD = 1024
N = 16384
D = 1024
NC, NS = 2, 16
NT = NC * NS
TCOLS = D // NT
K = 256
NCHUNK = N // K
CHUNKS_PER_S = NCHUNK // NS
CG = CPAD // 16
CGS = CG // NS


def _seg_body(emb, lab, refc, out_loss,
              rb0, rb1, ib0, ib1, acc_a, acc_b, cnt2, refv, sumb,
              invw_loc, invwv, lossb,
              sr0, si0, sr1, si1, sref,
              cnt_sh, invw_sh):
    c = lax.axis_index("c")
    s = lax.axis_index("s")
    t = c * NS + s
    col0 = t * TCOLS

    zvec = jnp.zeros((16,), jnp.float32)
    onevec = jnp.ones((16,), jnp.float32)

    for i in range(C, CPAD):
        refv[i, pl.ds(0, 16)] = zvec
        refv[i, pl.ds(16, 16)] = zvec
    pltpu.async_copy(refc.at[pl.ds(0, C), pl.ds(col0, TCOLS)],
                     refv.at[pl.ds(0, C)], sref)

    @plsc.parallel_loop(0, CPAD, unroll=4)
    def zbody(i):
        acc_a[i, :] = zvec
        acc_b[i, :] = zvec
    for i in range(CG):
        cnt2[i, :] = zvec

    def start(kc, rb, ib, sr, si):
        r0 = kc * K
        pltpu.async_copy(lab.at[pl.ds(r0, K)], ib, si)
        pltpu.async_copy(emb.at[pl.ds(r0, K), pl.ds(col0, TCOLS)], rb, sr)

    def wait(rb, ib, sr, si):
        pltpu.make_async_copy(lab.at[pl.ds(0, K)], ib, si).wait()
        pltpu.make_async_copy(emb.at[pl.ds(0, K), pl.ds(col0, TCOLS)], rb, sr).wait()

    def process(k, rb, ib):
        @plsc.parallel_loop(0, K // 16, unroll=4)
        def grp(g):
            j0 = g * 16
            lblv = ib[pl.ds(j0, 16)]
            lbls = [lblv[i] for i in range(16)]
            for i in range(16):
                plsc.addupdate(acc_a.at[lbls[i], :], rb[j0 + i, pl.ds(0, 16)])
            for i in range(16):
                plsc.addupdate(acc_b.at[lbls[i], :], rb[j0 + i, pl.ds(16, 16)])

        @pl.when((k >= s * CHUNKS_PER_S) & (k < (s + 1) * CHUNKS_PER_S))
        def _():
            def cgrp(g, _):
                lblv = ib[pl.ds(g * 16, 16)]
                plsc.addupdate_scatter(
                    cnt2, [lblv >> 4, lblv & 15], onevec)
                return 0
            lax.fori_loop(0, K // 16, cgrp, 0)

    start(0, rb0, ib0, sr0, si0)

    def outer(h, _):
        k0 = 2 * h
        start(jnp.minimum(k0 + 1, NCHUNK - 1), rb1, ib1, sr1, si1)
        wait(rb0, ib0, sr0, si0)
        process(k0, rb0, ib0)
        start(jnp.minimum(k0 + 2, NCHUNK - 1), rb0, ib0, sr0, si0)
        wait(rb1, ib1, sr1, si1)
        process(k0 + 1, rb1, ib1)
        return 0
    lax.fori_loop(0, NCHUNK // 2, outer, 0)

    wait(rb0, ib0, sr0, si0)

    pltpu.sync_copy(cnt2, cnt_sh.at[s])
    plsc.subcore_barrier()
    pltpu.sync_copy(cnt_sh.at[pl.ds(0, NS), pl.ds(s * CGS, CGS)], sumb)
    for r in range(CGS):
        tot = sumb[0, r, :]
        for p in range(1, NS):
            tot = tot + sumb[p, r, :]
        invw_loc[r, :] = 1.0 / jnp.maximum(tot, 1.0)
        invw_loc[CGS + r, :] = jnp.where(tot > 0.0, 1.0, 0.0)
    pltpu.sync_copy(invw_loc.at[pl.ds(0, CGS)], invw_sh.at[pl.ds(s * CGS, CGS)])
    pltpu.sync_copy(invw_loc.at[pl.ds(CGS, CGS)],
                    invw_sh.at[pl.ds(CG + s * CGS, CGS)])
    plsc.subcore_barrier()
    pltpu.sync_copy(invw_sh, invwv)

    pltpu.make_async_copy(
        refc.at[pl.ds(0, C), pl.ds(col0, TCOLS)],
        refv.at[pl.ds(0, C)], sref).wait()

    def loss_grp(g, carry):
        invv = invwv[g, :]
        wv = invwv[CG + g, :]
        for i in range(16):
            cc = g * 16 + i
            iv = jnp.broadcast_to(invv[i], (16,))
            wb = jnp.broadcast_to(wv[i], (16,))
            d0 = acc_a[cc, :] * iv - refv[cc, pl.ds(0, 16)]
            d1 = acc_b[cc, :] * iv - refv[cc, pl.ds(16, 16)]
            carry = carry + (d0 * d0 + d1 * d1) * wb
        return carry
    lossv = lax.fori_loop(0, CG, loss_grp, zvec, unroll=2)
    lossb[:] = lossv
    pltpu.sync_copy(lossb, out_loss.at[t])


@functools.lru_cache(maxsize=1)
def _make_seg_kernel():
    mesh = plsc.VectorSubcoreMesh(
        core_axis_name="c", subcore_axis_name="s",
        num_cores=NC, num_subcores=NS)
    return pl.kernel(
        _seg_body,
        out_type=jax.ShapeDtypeStruct((NT, 16), jnp.float32),
        mesh=mesh,
        compiler_params=pltpu.CompilerParams(
            use_tc_tiling_on_sc=False, needs_layout_passes=False),
        scratch_types=[
            pltpu.VMEM((K, TCOLS), jnp.float32),
            pltpu.VMEM((K, TCOLS), jnp.float32),
            pltpu.VMEM((K,), jnp.int32),
            pltpu.VMEM((K,), jnp.int32),
            pltpu.VMEM((CPAD, 16), jnp.float32),
            pltpu.VMEM((CPAD, 16), jnp.float32),
            pltpu.VMEM((CG, 16), jnp.float32),
            pltpu.VMEM((CPAD, TCOLS), jnp.float32),
            pltpu.VMEM((NS, CGS, 16), jnp.float32),
            pltpu.VMEM((2 * CGS, 16), jnp.float32),
            pltpu.VMEM((2 * CG, 16), jnp.float32),
            pltpu.VMEM((16,), jnp.float32),
            pltpu.SemaphoreType.DMA,
            pltpu.SemaphoreType.DMA,
            pltpu.SemaphoreType.DMA,
            pltpu.SemaphoreType.DMA,
            pltpu.SemaphoreType.DMA,
            pltpu.VMEM_SHARED((NS, CG, 16), jnp.float32),
            pltpu.VMEM_SHARED((2 * CG, 16), jnp.float32),
        ],
    )


def kernel(embeddings, labels, ref_centroids):
    seg = _make_seg_kernel()
    parts = seg(embeddings, labels, ref_centroids)
    return jnp.sum(parts) * (1.0 / D)


# speedup vs baseline: 1.4097x; 1.4097x over previous
import functools

import jax
import jax.numpy as jnp
from jax import lax
from jax.experimental import pallas as pl
from jax.experimental.pallas import tpu as pltpu
from jax.experimental.pallas import tpu_sc as plsc

C = 1000
CPAD = 1024
N = 16384
D = 1024
NC, NS = 2, 16
NT = NC * NS
TCOLS = D // NT
K = 256
NCHUNK = N // K
CHUNKS_PER_S = NCHUNK // NS
CG = CPAD // 16
CGS = CG // NS


def _seg_body(emb, lab, refc, out_loss,
              rb0, rb1, ib0, ib1, acc_a, acc_b, cnt2, refv, sumb,
              invw_loc, invwv, lossb,
              sr0, si0, sr1, si1, sref,
              cnt_sh, invw_sh):
    c = lax.axis_index("c")
    s = lax.axis_index("s")
    t = c * NS + s
    col0 = t * TCOLS

    zvec = jnp.zeros((16,), jnp.float32)
    onevec = jnp.ones((16,), jnp.float32)

    pltpu.async_copy(refc.at[pl.ds(0, CPAD), pl.ds(col0, TCOLS)], refv, sref)

    def zbody(i, _):
        acc_a[i, :] = zvec
        acc_b[i, :] = zvec
        return 0
    lax.fori_loop(0, CPAD, zbody, 0, unroll=4)
    for i in range(CG):
        cnt2[i, :] = zvec

    def start(kc, rb, ib, sr, si):
        r0 = kc * K
        pltpu.async_copy(lab.at[pl.ds(r0, K)], ib, si)
        pltpu.async_copy(emb.at[pl.ds(r0, K), pl.ds(col0, TCOLS)], rb, sr)

    def wait(rb, ib, sr, si):
        pltpu.make_async_copy(lab.at[pl.ds(0, K)], ib, si).wait()
        pltpu.make_async_copy(emb.at[pl.ds(0, K), pl.ds(col0, TCOLS)], rb, sr).wait()

    def process(k, rb, ib):
        @plsc.parallel_loop(0, K // 16, unroll=2)
        def grp(g):
            j0 = g * 16
            lblv = ib[pl.ds(j0, 16)]
            lbls = [lblv[i] for i in range(16)]
            for i in range(16):
                plsc.addupdate(acc_a.at[lbls[i], :], rb[j0 + i, pl.ds(0, 16)])
            for i in range(16):
                plsc.addupdate(acc_b.at[lbls[i], :], rb[j0 + i, pl.ds(16, 16)])

        @pl.when((k >= s * CHUNKS_PER_S) & (k < (s + 1) * CHUNKS_PER_S))
        def _():
            def cgrp(g, _):
                lblv = ib[pl.ds(g * 16, 16)]
                plsc.addupdate_scatter(
                    cnt2, [lblv >> 4, lblv & 15], onevec)
                return 0
            lax.fori_loop(0, K // 16, cgrp, 0)

    start(0, rb0, ib0, sr0, si0)

    def outer(h, _):
        k0 = 2 * h
        start(jnp.minimum(k0 + 1, NCHUNK - 1), rb1, ib1, sr1, si1)
        wait(rb0, ib0, sr0, si0)
        process(k0, rb0, ib0)
        start(jnp.minimum(k0 + 2, NCHUNK - 1), rb0, ib0, sr0, si0)
        wait(rb1, ib1, sr1, si1)
        process(k0 + 1, rb1, ib1)
        return 0
    lax.fori_loop(0, NCHUNK // 2, outer, 0)

    wait(rb0, ib0, sr0, si0)

    pltpu.sync_copy(cnt2, cnt_sh.at[s])
    plsc.subcore_barrier()
    pltpu.sync_copy(cnt_sh.at[pl.ds(0, NS), pl.ds(s * CGS, CGS)], sumb)
    for r in range(CGS):
        tot = sumb[0, r, :]
        for p in range(1, NS):
            tot = tot + sumb[p, r, :]
        invw_loc[r, :] = 1.0 / jnp.maximum(tot, 1.0)
        invw_loc[CGS + r, :] = jnp.where(tot > 0.0, 1.0, 0.0)
    pltpu.sync_copy(invw_loc.at[pl.ds(0, CGS)], invw_sh.at[pl.ds(s * CGS, CGS)])
    pltpu.sync_copy(invw_loc.at[pl.ds(CGS, CGS)],
                    invw_sh.at[pl.ds(CG + s * CGS, CGS)])
    plsc.subcore_barrier()
    pltpu.sync_copy(invw_sh, invwv)

    pltpu.make_async_copy(
        refc.at[pl.ds(0, CPAD), pl.ds(col0, TCOLS)], refv, sref).wait()

    def loss_grp(g, carry):
        invv = invwv[g, :]
        wv = invwv[CG + g, :]
        for i in range(16):
            cc = g * 16 + i
            iv = jnp.broadcast_to(invv[i], (16,))
            wb = jnp.broadcast_to(wv[i], (16,))
            d0 = acc_a[cc, :] * iv - refv[cc, pl.ds(0, 16)]
            d1 = acc_b[cc, :] * iv - refv[cc, pl.ds(16, 16)]
            carry = carry + (d0 * d0 + d1 * d1) * wb
        return carry
    lossv = lax.fori_loop(0, CG, loss_grp, zvec)
    lossb[:] = lossv
    pltpu.sync_copy(lossb, out_loss.at[t])


@functools.lru_cache(maxsize=1)
def _make_seg_kernel():
    mesh = plsc.VectorSubcoreMesh(
        core_axis_name="c", subcore_axis_name="s",
        num_cores=NC, num_subcores=NS)
    return pl.kernel(
        _seg_body,
        out_type=jax.ShapeDtypeStruct((NT, 16), jnp.float32),
        mesh=mesh,
        compiler_params=pltpu.CompilerParams(
            use_tc_tiling_on_sc=False, needs_layout_passes=False),
        scratch_types=[
            pltpu.VMEM((K, TCOLS), jnp.float32),
            pltpu.VMEM((K, TCOLS), jnp.float32),
            pltpu.VMEM((K,), jnp.int32),
            pltpu.VMEM((K,), jnp.int32),
            pltpu.VMEM((CPAD, 16), jnp.float32),
            pltpu.VMEM((CPAD, 16), jnp.float32),
            pltpu.VMEM((CG, 16), jnp.float32),
            pltpu.VMEM((CPAD, TCOLS), jnp.float32),
            pltpu.VMEM((NS, CGS, 16), jnp.float32),
            pltpu.VMEM((2 * CGS, 16), jnp.float32),
            pltpu.VMEM((2 * CG, 16), jnp.float32),
            pltpu.VMEM((16,), jnp.float32),
            pltpu.SemaphoreType.DMA,
            pltpu.SemaphoreType.DMA,
            pltpu.SemaphoreType.DMA,
            pltpu.SemaphoreType.DMA,
            pltpu.SemaphoreType.DMA,
            pltpu.VMEM_SHARED((NS, CG, 16), jnp.float32),
            pltpu.VMEM_SHARED((2 * CG, 16), jnp.float32),
        ],
    )


def kernel(embeddings, labels, ref_centroids):
    seg = _make_seg_kernel()
    refpad = jnp.pad(ref_centroids, ((0, CPAD - C), (0, 0)))
    parts = seg(embeddings, labels, refpad)
    return jnp.sum(parts) * (1.0 / D)
